# baseline (speedup 1.0000x reference)
import functools

import jax
import jax.numpy as jnp
from jax import lax
from jax.experimental import pallas as pl
from jax.experimental.pallas import tpu as pltpu

NE = 64
ND = 768
NA = 32
NN = 2048
NASS = 2 * NN
BLK = 128
NG = 96
NP = NG * BLK
GBLK = 256


def _elu(x):
    return jnp.where(x > 0, x, jnp.exp(jnp.minimum(x, 0.0)) - 1.0)


def _gating_body(obs_ref, w1_ref, b1_ref, w2_ref, b2_ref, w3_ref, b3_ref,
                 w_ref, idx_ref):
    h = _elu(jnp.dot(obs_ref[...], w1_ref[...],
                     preferred_element_type=jnp.float32) + b1_ref[...])
    h = _elu(jnp.dot(h, w2_ref[...],
                     preferred_element_type=jnp.float32) + b2_ref[...])
    lg = jnp.dot(h, w3_ref[...],
                 preferred_element_type=jnp.float32) + b3_ref[...]
    iota = lax.broadcasted_iota(jnp.int32, (GBLK, NE), 1)
    m1 = jnp.max(lg, axis=-1, keepdims=True)
    i1 = jnp.min(jnp.where(lg == m1, iota, NE), axis=-1, keepdims=True)
    masked = jnp.where(iota == i1, -1e30, lg)
    m2 = jnp.max(masked, axis=-1, keepdims=True)
    i2 = jnp.min(jnp.where(masked == m2, iota, NE), axis=-1, keepdims=True)
    d = jnp.exp(m2 - m1)
    s = 1.0 / (1.0 + d)
    w_ref[...] = jnp.concatenate([s, d * s], axis=-1)
    idx_ref[...] = jnp.concatenate([i1, i2], axis=-1)


def _gating(obs, g_W1, g_b1, g_W2, g_b2, g_W3, g_b3):
    return pl.pallas_call(
        _gating_body,
        grid=(NN // GBLK,),
        in_specs=[
            pl.BlockSpec((GBLK, ND), lambda g: (g, 0)),
            pl.BlockSpec((ND, 128), lambda g: (0, 0)),
            pl.BlockSpec((128,), lambda g: (0,)),
            pl.BlockSpec((128, 64), lambda g: (0, 0)),
            pl.BlockSpec((64,), lambda g: (0,)),
            pl.BlockSpec((64, NE), lambda g: (0, 0)),
            pl.BlockSpec((NE,), lambda g: (0,)),
        ],
        out_specs=[
            pl.BlockSpec((GBLK, 2), lambda g: (g, 0)),
            pl.BlockSpec((GBLK, 2), lambda g: (g, 0)),
        ],
        out_shape=[
            jax.ShapeDtypeStruct((NN, 2), jnp.float32),
            jax.ShapeDtypeStruct((NN, 2), jnp.int32),
        ],
    )(obs, g_W1, g_b1, g_W2, g_b2, g_W3, g_b3)


def _expert_body(bexp_ref, rows_ref, w1_ref, b1_ref, w2_ref, b2_ref,
                 w3_ref, b3_ref, out_ref):
    del bexp_ref
    h = _elu(jnp.dot(rows_ref[...], w1_ref[0],
                     preferred_element_type=jnp.float32) + b1_ref[0])
    h = _elu(jnp.dot(h, w2_ref[0],
                     preferred_element_type=jnp.float32) + b2_ref[0])
    out_ref[...] = jnp.dot(h, w3_ref[0],
                           preferred_element_type=jnp.float32) + b3_ref[0]


def _experts(rows, bexp, e_W1, e_b1, e_W2, e_b2, e_W3, e_b3):
    grid_spec = pltpu.PrefetchScalarGridSpec(
        num_scalar_prefetch=1,
        grid=(NG,),
        in_specs=[
            pl.BlockSpec((BLK, ND), lambda g, be: (g, 0)),
            pl.BlockSpec((1, ND, 256), lambda g, be: (be[g], 0, 0)),
            pl.BlockSpec((1, 1, 256), lambda g, be: (be[g], 0, 0)),
            pl.BlockSpec((1, 256, 128), lambda g, be: (be[g], 0, 0)),
            pl.BlockSpec((1, 1, 128), lambda g, be: (be[g], 0, 0)),
            pl.BlockSpec((1, 128, NA), lambda g, be: (be[g], 0, 0)),
            pl.BlockSpec((1, 1, NA), lambda g, be: (be[g], 0, 0)),
        ],
        out_specs=pl.BlockSpec((BLK, NA), lambda g, be: (g, 0)),
    )
    return pl.pallas_call(
        _expert_body,
        grid_spec=grid_spec,
        out_shape=jax.ShapeDtypeStruct((NP, NA), jnp.float32),
    )(bexp, rows, e_W1, e_b1[:, None, :], e_W2, e_b2[:, None, :],
      e_W3, e_b3[:, None, :])


def kernel(observations, g_W1, g_b1, g_W2, g_b2, g_W3, g_b3,
           e_W1, e_b1, e_W2, e_b2, e_W3, e_b3):
    topk_w, topk_idx = _gating(observations, g_W1, g_b1, g_W2, g_b2,
                               g_W3, g_b3)

    flat_e = topk_idx.reshape(-1)
    order = jnp.argsort(flat_e)
    counts = jnp.zeros((NE,), jnp.int32).at[flat_e].add(1)
    pc = ((counts + BLK - 1) // BLK) * BLK
    starts = jnp.cumsum(pc) - pc
    rawstarts = jnp.cumsum(counts) - counts
    e_sorted = flat_e[order]
    pos_sorted = starts[e_sorted] + (jnp.arange(NASS, dtype=jnp.int32)
                                     - rawstarts[e_sorted])
    inv = jnp.zeros((NASS,), jnp.int32).at[order].set(pos_sorted)
    src = jnp.zeros((NP,), jnp.int32).at[pos_sorted].set(order)
    rows = observations[src // 2]
    total = jnp.sum(pc)
    gb = jnp.minimum(jnp.arange(NG, dtype=jnp.int32) * BLK, total - BLK)
    bexp = (jnp.sum(starts[None, :] <= gb[:, None], axis=1) - 1).astype(
        jnp.int32)

    eout = _experts(rows, bexp, e_W1, e_b1, e_W2, e_b2, e_W3, e_b3)

    sel = eout[inv].reshape(NN, 2, NA)
    return jnp.sum(topk_w[:, :, None] * sel, axis=1)
